# baseline (speedup 1.0000x reference)
import jax
import jax.numpy as jnp
from jax import lax
from jax.experimental import pallas as pl
from jax.experimental.pallas import tpu as pltpu
from jax.experimental.pallas import tpu_sc as plsc

_N = 10000
_E = 320000
_D = 128
_EPS = 1e-5

_NC = 2
_NS = 16
_NW = _NC * _NS
_CH = 128
_NB = 2
_CPT = 80
_G = 16
_EPT = _CPT * _CH
_NPAD = 10112
_RPT = _NPAD // _NS


def _sc_agg_body(x_hbm, srcp, dstp, zrow, dep, agg_out,
                 src_v, dst_v, rows_v, agg_sh, gsem0, gsem1):
    del dep
    gsem = (gsem0, gsem1)
    c = lax.axis_index("c")
    s = lax.axis_index("s")
    wid = s * _NC + c
    pltpu.sync_copy(zrow, agg_sh.at[pl.ds(s * _RPT, _RPT)])
    plsc.subcore_barrier()

    def group(q, carry):
        pltpu.sync_copy(srcp.at[wid, pl.ds(q * _G, _G)], src_v)
        pltpu.sync_copy(dstp.at[wid, pl.ds(q * _G, _G)], dst_v)

        def pair(t, inner):
            pltpu.async_copy(x_hbm.at[src_v.at[t]], rows_v.at[0],
                             gsem0).wait()
            pltpu.sync_copy(rows_v.at[0], agg_sh.at[dst_v.at[t]], add=True)
            return inner

        lax.fori_loop(0, _G, pair, 0)
        return carry

    lax.fori_loop(0, _CPT // _G, group, 0)
    plsc.subcore_barrier()
    pltpu.sync_copy(agg_sh.at[pl.ds(s * _RPT, _RPT)],
                    agg_out.at[c, pl.ds(s * _RPT, _RPT)])


_sc_agg = pl.kernel(
    _sc_agg_body,
    out_type=jax.ShapeDtypeStruct((_NC, _NPAD, _D), jnp.float32),
    mesh=plsc.VectorSubcoreMesh(core_axis_name="c", subcore_axis_name="s"),
    scratch_types=[
        pltpu.VMEM((_G, _CH), jnp.int32),
        pltpu.VMEM((_G, _CH), jnp.int32),
        pltpu.VMEM((_NB, _CH, _D), jnp.float32),
        pltpu.VMEM_SHARED((_NPAD, _D), jnp.float32),
        pltpu.SemaphoreType.DMA,
        pltpu.SemaphoreType.DMA,
    ],
)


def _sc_cnt_body(dstp, zrow, ones_hbm, cnt_out, dst_v, ones_v, cnt_sh):
    c = lax.axis_index("c")
    s = lax.axis_index("s")
    wid = s * _NC + c
    pltpu.sync_copy(zrow, cnt_sh.at[pl.ds(s * _RPT, _RPT)])
    pltpu.sync_copy(ones_hbm, ones_v)
    plsc.subcore_barrier()

    def group(q, carry):
        pltpu.sync_copy(dstp.at[wid, pl.ds(q * _G, _G)], dst_v)

        def chunk(t, inner):
            pltpu.sync_copy(ones_v, cnt_sh.at[dst_v.at[t]], add=True)
            return inner

        lax.fori_loop(0, _G, chunk, 0)
        return carry

    lax.fori_loop(0, _CPT // _G, group, 0)
    plsc.subcore_barrier()
    pltpu.sync_copy(cnt_sh.at[pl.ds(s * _RPT, _RPT)],
                    cnt_out.at[c, pl.ds(s * _RPT, _RPT)])


_sc_cnt = pl.kernel(
    _sc_cnt_body,
    out_type=jax.ShapeDtypeStruct((_NC, _NPAD, _D), jnp.float32),
    mesh=plsc.VectorSubcoreMesh(core_axis_name="c", subcore_axis_name="s"),
    scratch_types=[
        pltpu.VMEM((_G, _CH), jnp.int32),
        pltpu.VMEM((_CH, _D), jnp.float32),
        pltpu.VMEM_SHARED((_NPAD, _D), jnp.float32),
    ],
)


def _dense_body(agg_ref, cnt_ref, x_ref, wl_ref, wr_ref, bl_ref, g_ref,
                b_ref, o_ref):
    agg = agg_ref[0, : _N, :] + agg_ref[1, : _N, :]
    cnt = cnt_ref[0, : _N, 0:1] + cnt_ref[1, : _N, 0:1]
    mean = agg / jnp.maximum(cnt, 1.0)
    x = x_ref[...]
    h = (jnp.dot(mean, wl_ref[...], preferred_element_type=jnp.float32)
         + jnp.dot(x, wr_ref[...], preferred_element_type=jnp.float32)
         + bl_ref[...])
    mu = jnp.mean(h, axis=0, keepdims=True)
    var = jnp.mean((h - mu) ** 2, axis=0, keepdims=True)
    hn = g_ref[...] * (h - mu) * lax.rsqrt(var + _EPS) + b_ref[...]
    o_ref[...] = x + jnp.maximum(hn, 0.0)


_dense = pl.pallas_call(
    _dense_body,
    out_shape=jax.ShapeDtypeStruct((_N, _D), jnp.float32),
)


def _dense_out_body(agg_ref, cnt_ref, x_ref, wl_ref, wr_ref, bl_ref, g_ref,
                    b_ref, wo_ref, bo_ref, o_ref):
    agg = agg_ref[0, : _N, :] + agg_ref[1, : _N, :]
    cnt = cnt_ref[0, : _N, 0:1] + cnt_ref[1, : _N, 0:1]
    mean = agg / jnp.maximum(cnt, 1.0)
    x = x_ref[...]
    h = (jnp.dot(mean, wl_ref[...], preferred_element_type=jnp.float32)
         + jnp.dot(x, wr_ref[...], preferred_element_type=jnp.float32)
         + bl_ref[...])
    mu = jnp.mean(h, axis=0, keepdims=True)
    var = jnp.mean((h - mu) ** 2, axis=0, keepdims=True)
    hn = g_ref[...] * (h - mu) * lax.rsqrt(var + _EPS) + b_ref[...]
    x3 = x + jnp.maximum(hn, 0.0)
    o_ref[...] = (jnp.dot(x3, wo_ref[...], preferred_element_type=jnp.float32)
                  + bo_ref[...])


_dense_out = pl.pallas_call(
    _dense_out_body,
    out_shape=jax.ShapeDtypeStruct((_N, 40), jnp.float32),
)


def kernel(x, edge_index, Wl1, bl1, Wr1, g1, b1, Wl2, bl2, Wr2, g2, b2,
           Wl3, bl3, Wr3, g3, b3, Wout, bout):
    src = edge_index[0]
    dst = edge_index[1]
    pad = _NW * _EPT - _E
    srcp = jnp.concatenate([src, jnp.zeros((pad,), jnp.int32)])
    srcp = srcp.reshape(_NW, _CPT, _CH)
    dummy = _N + jnp.arange(pad, dtype=jnp.int32) % (_NPAD - _N)
    dstp = jnp.concatenate([dst, dummy])
    dstp = dstp.reshape(_NW, _CPT, _CH)
    zrow = jnp.zeros((_RPT, _D), jnp.float32)
    ones = jnp.ones((_CH, _D), jnp.float32)

    cnt = _sc_cnt(dstp, zrow, ones)

    h = x
    layers = [(Wl1, bl1, Wr1, g1, b1), (Wl2, bl2, Wr2, g2, b2),
              (Wl3, bl3, Wr3, g3, b3)]
    for i, (Wl, bl, Wr, g, b) in enumerate(layers):
        agg = _sc_agg(h, srcp, dstp, zrow, cnt)
        if i < 2:
            h = _dense(agg, cnt, h, Wl.T, Wr.T, bl[None, :], g[None, :],
                       b[None, :])
        else:
            out = _dense_out(agg, cnt, h, Wl.T, Wr.T, bl[None, :],
                             g[None, :], b[None, :], Wout.T, bout[None, :])
    return out


# speedup vs baseline: 1.1061x; 1.1061x over previous
import jax
import jax.numpy as jnp
from jax import lax
from jax.experimental import pallas as pl
from jax.experimental.pallas import tpu as pltpu
from jax.experimental.pallas import tpu_sc as plsc

_N = 10000
_E = 320000
_D = 128
_EPS = 1e-5

_NC = 2
_NS = 16
_NW = _NC * _NS
_CH = 128
_NB = 2
_CPT = 80
_G = 16
_EPT = _CPT * _CH
_NPAD = 10112
_RPT = _NPAD // _NS


def _sc_agg_body(x_hbm, srcp, dstp, zrow, agg_out,
                 src_v, dst_v, rows_v, agg_sh, sem):
    c = lax.axis_index("c")
    s = lax.axis_index("s")
    wid = s * _NC + c
    pltpu.sync_copy(zrow, agg_sh.at[pl.ds(s * _RPT, _RPT)])
    pltpu.sync_copy(srcp.at[wid], src_v)
    pltpu.sync_copy(dstp.at[wid], dst_v)
    plsc.subcore_barrier()

    def chunk(j, carry):
        pltpu.async_copy(x_hbm.at[src_v.at[j]], rows_v, sem).wait()
        pltpu.sync_copy(rows_v, agg_sh.at[dst_v.at[j]], add=True)
        return carry

    lax.fori_loop(0, _CPT, chunk, 0)
    plsc.subcore_barrier()
    pltpu.sync_copy(agg_sh.at[pl.ds(s * _RPT, _RPT)],
                    agg_out.at[c, pl.ds(s * _RPT, _RPT)])


_sc_agg = pl.kernel(
    _sc_agg_body,
    out_type=jax.ShapeDtypeStruct((_NC, _NPAD, _D), jnp.float32),
    mesh=plsc.VectorSubcoreMesh(core_axis_name="c", subcore_axis_name="s"),
    scratch_types=[
        pltpu.VMEM((_CPT, _CH), jnp.int32),
        pltpu.VMEM((_CPT, _CH), jnp.int32),
        pltpu.VMEM((_CH, _D), jnp.float32),
        pltpu.VMEM_SHARED((_NPAD, _D), jnp.float32),
        pltpu.SemaphoreType.DMA,
    ],
)


def _sc_cnt_body(dstp, zrow, ones_hbm, cnt_out, dst_v, ones_v, cnt_sh):
    c = lax.axis_index("c")
    s = lax.axis_index("s")
    wid = s * _NC + c
    pltpu.sync_copy(zrow, cnt_sh.at[pl.ds(s * _RPT, _RPT)])
    pltpu.sync_copy(ones_hbm, ones_v)
    plsc.subcore_barrier()

    def group(q, carry):
        pltpu.sync_copy(dstp.at[wid, pl.ds(q * _G, _G)], dst_v)

        def chunk(t, inner):
            pltpu.sync_copy(ones_v, cnt_sh.at[dst_v.at[t]], add=True)
            return inner

        lax.fori_loop(0, _G, chunk, 0)
        return carry

    lax.fori_loop(0, _CPT // _G, group, 0)
    plsc.subcore_barrier()
    pltpu.sync_copy(cnt_sh.at[pl.ds(s * _RPT, _RPT)],
                    cnt_out.at[c, pl.ds(s * _RPT, _RPT)])


_sc_cnt = pl.kernel(
    _sc_cnt_body,
    out_type=jax.ShapeDtypeStruct((_NC, _NPAD, _D), jnp.float32),
    mesh=plsc.VectorSubcoreMesh(core_axis_name="c", subcore_axis_name="s"),
    scratch_types=[
        pltpu.VMEM((_G, _CH), jnp.int32),
        pltpu.VMEM((_CH, _D), jnp.float32),
        pltpu.VMEM_SHARED((_NPAD, _D), jnp.float32),
    ],
)


def _dense_body(agg_ref, cnt_ref, x_ref, wl_ref, wr_ref, bl_ref, g_ref,
                b_ref, o_ref):
    agg = agg_ref[0, : _N, :] + agg_ref[1, : _N, :]
    cnt = cnt_ref[0, : _N, 0:1] + cnt_ref[1, : _N, 0:1]
    mean = agg / jnp.maximum(cnt, 1.0)
    x = x_ref[...]
    h = (jnp.dot(mean, wl_ref[...], preferred_element_type=jnp.float32)
         + jnp.dot(x, wr_ref[...], preferred_element_type=jnp.float32)
         + bl_ref[...])
    mu = jnp.mean(h, axis=0, keepdims=True)
    var = jnp.mean((h - mu) ** 2, axis=0, keepdims=True)
    hn = g_ref[...] * (h - mu) * lax.rsqrt(var + _EPS) + b_ref[...]
    o_ref[...] = x + jnp.maximum(hn, 0.0)


_dense = pl.pallas_call(
    _dense_body,
    out_shape=jax.ShapeDtypeStruct((_N, _D), jnp.float32),
)


def _dense_out_body(agg_ref, cnt_ref, x_ref, wl_ref, wr_ref, bl_ref, g_ref,
                    b_ref, wo_ref, bo_ref, o_ref):
    agg = agg_ref[0, : _N, :] + agg_ref[1, : _N, :]
    cnt = cnt_ref[0, : _N, 0:1] + cnt_ref[1, : _N, 0:1]
    mean = agg / jnp.maximum(cnt, 1.0)
    x = x_ref[...]
    h = (jnp.dot(mean, wl_ref[...], preferred_element_type=jnp.float32)
         + jnp.dot(x, wr_ref[...], preferred_element_type=jnp.float32)
         + bl_ref[...])
    mu = jnp.mean(h, axis=0, keepdims=True)
    var = jnp.mean((h - mu) ** 2, axis=0, keepdims=True)
    hn = g_ref[...] * (h - mu) * lax.rsqrt(var + _EPS) + b_ref[...]
    x3 = x + jnp.maximum(hn, 0.0)
    o_ref[...] = (jnp.dot(x3, wo_ref[...], preferred_element_type=jnp.float32)
                  + bo_ref[...])


_dense_out = pl.pallas_call(
    _dense_out_body,
    out_shape=jax.ShapeDtypeStruct((_N, 40), jnp.float32),
)


def kernel(x, edge_index, Wl1, bl1, Wr1, g1, b1, Wl2, bl2, Wr2, g2, b2,
           Wl3, bl3, Wr3, g3, b3, Wout, bout):
    src = edge_index[0]
    dst = edge_index[1]
    pad = _NW * _EPT - _E
    srcp = jnp.concatenate([src, jnp.zeros((pad,), jnp.int32)])
    srcp = srcp.reshape(_NW, _CPT, _CH)
    dummy = _N + jnp.arange(pad, dtype=jnp.int32) % (_NPAD - _N)
    dstp = jnp.concatenate([dst, dummy])
    dstp = dstp.reshape(_NW, _CPT, _CH)
    zrow = jnp.zeros((_RPT, _D), jnp.float32)
    ones = jnp.ones((_CH, _D), jnp.float32)

    cnt = _sc_cnt(dstp, zrow, ones)

    h = x
    layers = [(Wl1, bl1, Wr1, g1, b1), (Wl2, bl2, Wr2, g2, b2),
              (Wl3, bl3, Wr3, g3, b3)]
    for i, (Wl, bl, Wr, g, b) in enumerate(layers):
        agg = _sc_agg(h, srcp, dstp, zrow)
        if i < 2:
            h = _dense(agg, cnt, h, Wl.T, Wr.T, bl[None, :], g[None, :],
                       b[None, :])
        else:
            out = _dense_out(agg, cnt, h, Wl.T, Wr.T, bl[None, :],
                             g[None, :], b[None, :], Wout.T, bout[None, :])
    return out


# speedup vs baseline: 1.1085x; 1.0022x over previous
import jax
import jax.numpy as jnp
from jax import lax
from jax.experimental import pallas as pl
from jax.experimental.pallas import tpu as pltpu
from jax.experimental.pallas import tpu_sc as plsc

_N = 10000
_E = 320000
_D = 128
_EPS = 1e-5

_NC = 2
_NS = 16
_NW = _NC * _NS
_CH = 128
_NB = 2
_CPT = 80
_G = 16
_EPT = _CPT * _CH
_NPAD = 10112
_RPT = _NPAD // _NS


def _sc_agg_body(x_hbm, srcp, dstp, zrow, agg_out,
                 src_v, dst_v, rows_v, agg_sh, sem):
    c = lax.axis_index("c")
    s = lax.axis_index("s")
    wid = s * _NC + c
    pltpu.sync_copy(zrow, agg_sh.at[pl.ds(s * _RPT, _RPT)])
    pltpu.sync_copy(srcp.at[wid], src_v)
    pltpu.sync_copy(dstp.at[wid], dst_v)
    plsc.subcore_barrier()

    def chunk(j, carry):
        pltpu.async_copy(x_hbm.at[src_v.at[j]], rows_v, sem).wait()
        pltpu.sync_copy(rows_v, agg_sh.at[dst_v.at[j]], add=True)
        return carry

    lax.fori_loop(0, _CPT, chunk, 0)
    plsc.subcore_barrier()
    pltpu.sync_copy(agg_sh.at[pl.ds(s * _RPT, _RPT)],
                    agg_out.at[c, pl.ds(s * _RPT, _RPT)])


_sc_agg = pl.kernel(
    _sc_agg_body,
    out_type=jax.ShapeDtypeStruct((_NC, _NPAD, _D), jnp.float32),
    mesh=plsc.VectorSubcoreMesh(core_axis_name="c", subcore_axis_name="s"),
    scratch_types=[
        pltpu.VMEM((_CPT, _CH), jnp.int32),
        pltpu.VMEM((_CPT, _CH), jnp.int32),
        pltpu.VMEM((_CH, _D), jnp.float32),
        pltpu.VMEM_SHARED((_NPAD, _D), jnp.float32),
        pltpu.SemaphoreType.DMA,
    ],
)


def _sc_cnt_body(dstp, zrow, ones_hbm, cnt_out, dst_v, ones_v, cnt_sh):
    c = lax.axis_index("c")
    s = lax.axis_index("s")
    wid = s * _NC + c
    pltpu.sync_copy(zrow, cnt_sh.at[pl.ds(s * _RPT, _RPT)])
    pltpu.sync_copy(dstp.at[wid], dst_v)
    pltpu.sync_copy(ones_hbm, ones_v)
    plsc.subcore_barrier()

    def chunk(j, carry):
        pltpu.sync_copy(ones_v, cnt_sh.at[dst_v.at[j]], add=True)
        return carry

    lax.fori_loop(0, _CPT, chunk, 0)
    plsc.subcore_barrier()
    pltpu.sync_copy(cnt_sh.at[pl.ds(s * _RPT, _RPT)],
                    cnt_out.at[c, pl.ds(s * _RPT, _RPT)])


_sc_cnt = pl.kernel(
    _sc_cnt_body,
    out_type=jax.ShapeDtypeStruct((_NC, _NPAD, _D), jnp.float32),
    mesh=plsc.VectorSubcoreMesh(core_axis_name="c", subcore_axis_name="s"),
    scratch_types=[
        pltpu.VMEM((_CPT, _CH), jnp.int32),
        pltpu.VMEM((_CH, _D), jnp.float32),
        pltpu.VMEM_SHARED((_NPAD, _D), jnp.float32),
    ],
)


def _dense_body(agg_ref, cnt_ref, x_ref, wl_ref, wr_ref, bl_ref, g_ref,
                b_ref, o_ref):
    agg = agg_ref[0, : _N, :] + agg_ref[1, : _N, :]
    cnt = cnt_ref[0, : _N, 0:1] + cnt_ref[1, : _N, 0:1]
    mean = agg / jnp.maximum(cnt, 1.0)
    x = x_ref[...]
    h = (jnp.dot(mean, wl_ref[...], preferred_element_type=jnp.float32)
         + jnp.dot(x, wr_ref[...], preferred_element_type=jnp.float32)
         + bl_ref[...])
    mu = jnp.mean(h, axis=0, keepdims=True)
    var = jnp.mean((h - mu) ** 2, axis=0, keepdims=True)
    hn = g_ref[...] * (h - mu) * lax.rsqrt(var + _EPS) + b_ref[...]
    o_ref[...] = x + jnp.maximum(hn, 0.0)


_dense = pl.pallas_call(
    _dense_body,
    out_shape=jax.ShapeDtypeStruct((_N, _D), jnp.float32),
)


def _dense_out_body(agg_ref, cnt_ref, x_ref, wl_ref, wr_ref, bl_ref, g_ref,
                    b_ref, wo_ref, bo_ref, o_ref):
    agg = agg_ref[0, : _N, :] + agg_ref[1, : _N, :]
    cnt = cnt_ref[0, : _N, 0:1] + cnt_ref[1, : _N, 0:1]
    mean = agg / jnp.maximum(cnt, 1.0)
    x = x_ref[...]
    h = (jnp.dot(mean, wl_ref[...], preferred_element_type=jnp.float32)
         + jnp.dot(x, wr_ref[...], preferred_element_type=jnp.float32)
         + bl_ref[...])
    mu = jnp.mean(h, axis=0, keepdims=True)
    var = jnp.mean((h - mu) ** 2, axis=0, keepdims=True)
    hn = g_ref[...] * (h - mu) * lax.rsqrt(var + _EPS) + b_ref[...]
    x3 = x + jnp.maximum(hn, 0.0)
    o_ref[...] = (jnp.dot(x3, wo_ref[...], preferred_element_type=jnp.float32)
                  + bo_ref[...])


_dense_out = pl.pallas_call(
    _dense_out_body,
    out_shape=jax.ShapeDtypeStruct((_N, 40), jnp.float32),
)


def kernel(x, edge_index, Wl1, bl1, Wr1, g1, b1, Wl2, bl2, Wr2, g2, b2,
           Wl3, bl3, Wr3, g3, b3, Wout, bout):
    src = edge_index[0]
    dst = edge_index[1]
    pad = _NW * _EPT - _E
    srcp = jnp.concatenate([src, jnp.zeros((pad,), jnp.int32)])
    srcp = srcp.reshape(_NW, _CPT, _CH)
    dummy = _N + jnp.arange(pad, dtype=jnp.int32) % (_NPAD - _N)
    dstp = jnp.concatenate([dst, dummy])
    dstp = dstp.reshape(_NW, _CPT, _CH)
    zrow = jnp.zeros((_RPT, _D), jnp.float32)
    ones = jnp.ones((_CH, _D), jnp.float32)

    cnt = _sc_cnt(dstp, zrow, ones)

    h = x
    layers = [(Wl1, bl1, Wr1, g1, b1), (Wl2, bl2, Wr2, g2, b2),
              (Wl3, bl3, Wr3, g3, b3)]
    for i, (Wl, bl, Wr, g, b) in enumerate(layers):
        agg = _sc_agg(h, srcp, dstp, zrow)
        if i < 2:
            h = _dense(agg, cnt, h, Wl.T, Wr.T, bl[None, :], g[None, :],
                       b[None, :])
        else:
            out = _dense_out(agg, cnt, h, Wl.T, Wr.T, bl[None, :],
                             g[None, :], b[None, :], Wout.T, bout[None, :])
    return out


# speedup vs baseline: 2.6652x; 2.4044x over previous
import jax
import jax.numpy as jnp
from jax import lax
from jax.experimental import pallas as pl
from jax.experimental.pallas import tpu as pltpu
from jax.experimental.pallas import tpu_sc as plsc

_N = 10000
_E = 320000
_D = 128
_EPS = 1e-5

_NC = 2
_NS = 16
_NW = _NC * _NS
_CH = 128
_NB = 2
_CPT = 80
_G = 16
_EPT = _CPT * _CH
_NPAD = 10112
_RPT = _NPAD // _NS


def _sc_agg_body(x_hbm, srcp, dstp, zrow, agg_out,
                 src_v, dst_v, rows_v, agg_sh, sem):
    c = lax.axis_index("c")
    s = lax.axis_index("s")
    wid = s * _NC + c
    pltpu.sync_copy(zrow, agg_sh.at[pl.ds(s * _RPT, _RPT)])
    pltpu.sync_copy(srcp.at[wid], src_v)
    pltpu.sync_copy(dstp.at[wid], dst_v)
    plsc.subcore_barrier()

    def chunk(j, carry):
        pltpu.async_copy(x_hbm.at[src_v.at[j]], rows_v, sem).wait()
        pltpu.sync_copy(rows_v, agg_sh.at[dst_v.at[j]], add=True)
        return carry

    lax.fori_loop(0, _CPT, chunk, 0)
    plsc.subcore_barrier()
    pltpu.sync_copy(agg_sh.at[pl.ds(s * _RPT, _RPT)],
                    agg_out.at[c, pl.ds(s * _RPT, _RPT)])


_sc_agg = pl.kernel(
    _sc_agg_body,
    out_type=jax.ShapeDtypeStruct((_NC, _NPAD, _D), jnp.float32),
    mesh=plsc.VectorSubcoreMesh(core_axis_name="c", subcore_axis_name="s"),
    scratch_types=[
        pltpu.VMEM((_CPT, _CH), jnp.int32),
        pltpu.VMEM((_CPT, _CH), jnp.int32),
        pltpu.VMEM((_CH, _D), jnp.float32),
        pltpu.VMEM_SHARED((_NPAD, _D), jnp.float32),
        pltpu.SemaphoreType.DMA,
    ],
)


def _sc_cnt_body(dstp, zrow, ones_hbm, cnt_out, dst_v, ones_v, cnt_sh):
    c = lax.axis_index("c")
    s = lax.axis_index("s")
    wid = s * _NC + c
    pltpu.sync_copy(zrow, cnt_sh.at[pl.ds(s * _RPT, _RPT)])
    pltpu.sync_copy(dstp.at[wid], dst_v)
    pltpu.sync_copy(ones_hbm, ones_v)
    plsc.subcore_barrier()

    def chunk(j, carry):
        pltpu.sync_copy(ones_v, cnt_sh.at[dst_v.at[j]], add=True)
        return carry

    lax.fori_loop(0, _CPT, chunk, 0)
    plsc.subcore_barrier()
    pltpu.sync_copy(cnt_sh.at[pl.ds(s * _RPT, _RPT)],
                    cnt_out.at[c, pl.ds(s * _RPT, _RPT)])


_sc_cnt = pl.kernel(
    _sc_cnt_body,
    out_type=jax.ShapeDtypeStruct((_NC, _NPAD, _D), jnp.float32),
    mesh=plsc.VectorSubcoreMesh(core_axis_name="c", subcore_axis_name="s"),
    scratch_types=[
        pltpu.VMEM((_CPT, _CH), jnp.int32),
        pltpu.VMEM((_CH, _D), jnp.float32),
        pltpu.VMEM_SHARED((_NPAD, _D), jnp.float32),
    ],
)


def _dense_body(agg_ref, cnt_ref, x_ref, wl_ref, wr_ref, bl_ref, g_ref,
                b_ref, o_ref):
    agg = agg_ref[0, : _N, :] + agg_ref[1, : _N, :]
    cnt = cnt_ref[0, : _N, 0:1] + cnt_ref[1, : _N, 0:1]
    mean = agg / jnp.maximum(cnt, 1.0)
    x = x_ref[...]
    h = (jnp.dot(mean, wl_ref[...], preferred_element_type=jnp.float32)
         + jnp.dot(x, wr_ref[...], preferred_element_type=jnp.float32)
         + bl_ref[...])
    mu = jnp.mean(h, axis=0, keepdims=True)
    var = jnp.mean((h - mu) ** 2, axis=0, keepdims=True)
    hn = g_ref[...] * (h - mu) * lax.rsqrt(var + _EPS) + b_ref[...]
    o_ref[...] = x + jnp.maximum(hn, 0.0)


_dense = pl.pallas_call(
    _dense_body,
    out_shape=jax.ShapeDtypeStruct((_N, _D), jnp.float32),
)


def _dense_out_body(agg_ref, cnt_ref, x_ref, wl_ref, wr_ref, bl_ref, g_ref,
                    b_ref, wo_ref, bo_ref, o_ref):
    agg = agg_ref[0, : _N, :] + agg_ref[1, : _N, :]
    cnt = cnt_ref[0, : _N, 0:1] + cnt_ref[1, : _N, 0:1]
    mean = agg / jnp.maximum(cnt, 1.0)
    x = x_ref[...]
    h = (jnp.dot(mean, wl_ref[...], preferred_element_type=jnp.float32)
         + jnp.dot(x, wr_ref[...], preferred_element_type=jnp.float32)
         + bl_ref[...])
    mu = jnp.mean(h, axis=0, keepdims=True)
    var = jnp.mean((h - mu) ** 2, axis=0, keepdims=True)
    hn = g_ref[...] * (h - mu) * lax.rsqrt(var + _EPS) + b_ref[...]
    x3 = x + jnp.maximum(hn, 0.0)
    o_ref[...] = (jnp.dot(x3, wo_ref[...], preferred_element_type=jnp.float32)
                  + bo_ref[...])


_dense_out = pl.pallas_call(
    _dense_out_body,
    out_shape=jax.ShapeDtypeStruct((_N, 40), jnp.float32),
)


def kernel(x, edge_index, Wl1, bl1, Wr1, g1, b1, Wl2, bl2, Wr2, g2, b2,
           Wl3, bl3, Wr3, g3, b3, Wout, bout):
    src = edge_index[0]
    dst = edge_index[1]
    pad = _NW * _EPT - _E
    dummy_src = (jnp.arange(pad, dtype=jnp.int32) * 131) % _N
    srcp = jnp.concatenate([src, dummy_src])
    srcp = srcp.reshape(_NW, _CPT, _CH)
    dummy = _N + jnp.arange(pad, dtype=jnp.int32) % (_NPAD - _N)
    dstp = jnp.concatenate([dst, dummy])
    dstp = dstp.reshape(_NW, _CPT, _CH)
    zrow = jnp.zeros((_RPT, _D), jnp.float32)
    ones = jnp.ones((_CH, _D), jnp.float32)

    cnt = _sc_cnt(dstp, zrow, ones)

    h = x
    layers = [(Wl1, bl1, Wr1, g1, b1), (Wl2, bl2, Wr2, g2, b2),
              (Wl3, bl3, Wr3, g3, b3)]
    for i, (Wl, bl, Wr, g, b) in enumerate(layers):
        agg = _sc_agg(h, srcp, dstp, zrow)
        if i < 2:
            h = _dense(agg, cnt, h, Wl.T, Wr.T, bl[None, :], g[None, :],
                       b[None, :])
        else:
            out = _dense_out(agg, cnt, h, Wl.T, Wr.T, bl[None, :],
                             g[None, :], b[None, :], Wout.T, bout[None, :])
    return out
